# Initial kernel scaffold; baseline (speedup 1.0000x reference)
import jax
import jax.numpy as jnp
from jax.experimental import pallas as pl


def kernel(x, edge_index, edge_attr, pair_indices, W1, b1, W2, b2, Wlin, blin):
    raise NotImplementedError("write your pallas kernel here")


# speedup vs baseline: 28.6392x; 28.6392x over previous
import functools

import jax
import jax.numpy as jnp
from jax import lax
from jax.experimental import pallas as pl
from jax.experimental.pallas import tpu as pltpu
from jax.experimental.pallas import tpu_sc as plsc

N = 10000
E = 320000
H = 128
P = 100000

NC = 2
NS = 16
NW = NC * NS
EPT = E // NW
CHUNK = 80
NCHUNK = EPT // CHUNK
ZROW = 640
LROW = N - (NS - 1) * ZROW
DEGW = 16
PC = 160
NPC = P // PC
PK = (NPC + NW - 1) // NW

_mesh = plsc.VectorSubcoreMesh(core_axis_name="c", subcore_axis_name="s")


def _zero_slice(zeros_hbm, acc_sh, s):
    @pl.when(s < NS - 1)
    def _():
        pltpu.sync_copy(zeros_hbm, acc_sh.at[pl.ds(s * ZROW, ZROW)])

    @pl.when(s == NS - 1)
    def _():
        pltpu.sync_copy(zeros_hbm.at[pl.ds(0, LROW)],
                        acc_sh.at[pl.ds((NS - 1) * ZROW, LROW)])


def _copy_out(acc_sh, out_hbm, c, s):
    @pl.when(s < NS - 1)
    def _():
        pltpu.sync_copy(acc_sh.at[pl.ds(s * ZROW, ZROW)],
                        out_hbm.at[c, pl.ds(s * ZROW, ZROW)])

    @pl.when(s == NS - 1)
    def _():
        pltpu.sync_copy(acc_sh.at[pl.ds((NS - 1) * ZROW, LROW)],
                        out_hbm.at[c, pl.ds((NS - 1) * ZROW, LROW)])


def _deg_body(dst_hbm, zeros_hbm, ones_hbm, out_hbm, idx_v, ones_v, stg_v,
              acc_sh):
    c = lax.axis_index("c")
    s = lax.axis_index("s")
    wid = c * NS + s
    pltpu.sync_copy(dst_hbm.at[wid], idx_v)
    pltpu.sync_copy(ones_hbm, ones_v)

    pltpu.sync_copy(zeros_hbm, stg_v)

    @pl.when(s < NS - 1)
    def _():
        pltpu.sync_copy(stg_v, acc_sh.at[pl.ds(s * ZROW, ZROW)])

    @pl.when(s == NS - 1)
    def _():
        pltpu.sync_copy(stg_v.at[pl.ds(0, LROW)],
                        acc_sh.at[pl.ds((NS - 1) * ZROW, LROW)])

    plsc.subcore_barrier()

    def body(ci, carry):
        pltpu.sync_copy(ones_v, acc_sh.at[idx_v.at[ci, 0]], add=True)
        return carry

    lax.fori_loop(0, NCHUNK, body, 0)
    plsc.subcore_barrier()

    @pl.when(s < NS - 1)
    def _():
        pltpu.sync_copy(acc_sh.at[pl.ds(s * ZROW, ZROW)], stg_v)
        pltpu.sync_copy(stg_v, out_hbm.at[pl.ds(c * N + s * ZROW, ZROW)])

    @pl.when(s == NS - 1)
    def _():
        pltpu.sync_copy(acc_sh.at[pl.ds((NS - 1) * ZROW, LROW)],
                        stg_v.at[pl.ds(0, LROW)])
        pltpu.sync_copy(stg_v.at[pl.ds(0, LROW)],
                        out_hbm.at[pl.ds(c * N + (NS - 1) * ZROW, LROW)])


NBUF = 3
NIDX = 2 * NBUF


def _agg_body(src_hbm, dst_hbm, g_hbm, zeros_hbm, out_hbm,
                sidx_v, didx_v, rows_v, acc_sh, semg, semis, semid):
    c = lax.axis_index("c")
    s = lax.axis_index("s")
    wid = c * NS + s
    _zero_slice(zeros_hbm, acc_sh, s)

    for j in range(NIDX):
        pltpu.async_copy(src_hbm.at[wid, j], sidx_v.at[j], semis.at[j])
    for b in range(NBUF):
        pltpu.async_copy(dst_hbm.at[wid, b], didx_v.at[b], semid.at[b])
        pltpu.make_async_copy(src_hbm.at[wid, 0], sidx_v.at[b],
                              semis.at[b]).wait()
        pltpu.async_copy(g_hbm.at[sidx_v.at[b, 0]], rows_v.at[b], semg.at[b])
    plsc.subcore_barrier()

    def body(ci, carry):
        b = lax.rem(ci, NBUF)
        pltpu.make_async_copy(dst_hbm.at[wid, 0], didx_v.at[b],
                              semid.at[b]).wait()
        pltpu.make_async_copy(g_hbm.at[sidx_v.at[0, 0]], rows_v.at[b],
                              semg.at[b]).wait()
        pltpu.sync_copy(rows_v.at[b], acc_sh.at[didx_v.at[b, 0]], add=True)
        nxt = ci + NBUF

        @pl.when(nxt < NCHUNK)
        def _():
            bn = lax.rem(nxt, NIDX)
            pltpu.make_async_copy(src_hbm.at[wid, 0], sidx_v.at[bn],
                                  semis.at[bn]).wait()
            pltpu.async_copy(g_hbm.at[sidx_v.at[bn, 0]], rows_v.at[b],
                             semg.at[b])
            pltpu.async_copy(dst_hbm.at[wid, nxt], didx_v.at[b], semid.at[b])

        nxt2 = ci + NIDX

        @pl.when(nxt2 < NCHUNK)
        def _():
            b2 = lax.rem(ci, NIDX)
            pltpu.async_copy(src_hbm.at[wid, nxt2], sidx_v.at[b2],
                             semis.at[b2])

        return carry

    lax.fori_loop(0, NCHUNK, body, 0)
    plsc.subcore_barrier()
    _copy_out(acc_sh, out_hbm, c, s)


def _pair_body(s_hbm, t_hbm, i0_hbm, i1_hbm, out_hbm,
                 s_v, t_v, i0_v, i1_v, ob_v):
    c = lax.axis_index("c")
    s = lax.axis_index("s")
    wid = c * NS + s
    pltpu.sync_copy(s_hbm, s_v)
    pltpu.sync_copy(t_hbm, t_v)
    for k in range(PK):
        cid = wid + NW * k

        @pl.when(cid < NPC)
        def _():
            base = cid * PC
            pltpu.sync_copy(i0_hbm.at[pl.ds(base, PC)], i0_v)
            pltpu.sync_copy(i1_hbm.at[pl.ds(base, PC)], i1_v)

            def body(j, carry):
                idx0 = i0_v[pl.ds(j * 16, 16)]
                idx1 = i1_v[pl.ds(j * 16, 16)]
                v0 = plsc.load_gather(s_v, [idx0])
                v1 = plsc.load_gather(t_v, [idx1])
                z = v0 + v1
                ob_v[pl.ds(j * 16, 16)] = 1.0 / (1.0 + jnp.exp(-z))
                return carry

            lax.fori_loop(0, PC // 16, body, 0)
            pltpu.sync_copy(ob_v, out_hbm.at[pl.ds(base, PC)])


def _mk_deg_kernel(interpret=False):
    return pl.kernel(
        _deg_body,
        out_type=jax.ShapeDtypeStruct((NC * N,), jnp.float32),
        mesh=_mesh,
        interpret=interpret,
        scratch_types=[
            pltpu.VMEM((NCHUNK, 1, CHUNK), jnp.int32),
            pltpu.VMEM((CHUNK,), jnp.float32),
            pltpu.VMEM((ZROW,), jnp.float32),
            pltpu.VMEM_SHARED((N,), jnp.float32),
        ],
    )


def _mk_agg_kernel(interpret=False):
    return pl.kernel(
        _agg_body,
        out_type=jax.ShapeDtypeStruct((NC, N, H), jnp.float32),
        mesh=_mesh,
        interpret=interpret,
        scratch_types=[
            pltpu.VMEM((NIDX, 1, CHUNK), jnp.int32),
            pltpu.VMEM((NBUF, 1, CHUNK), jnp.int32),
            pltpu.VMEM((NBUF, CHUNK, H), jnp.float32),
            pltpu.VMEM_SHARED((N, H), jnp.float32),
            pltpu.SemaphoreType.DMA((NBUF,)),
            pltpu.SemaphoreType.DMA((NIDX,)),
            pltpu.SemaphoreType.DMA((NBUF,)),
        ],
    )


def _mk_pair_kernel(interpret=False):
    return pl.kernel(
        _pair_body,
        out_type=jax.ShapeDtypeStruct((P,), jnp.float32),
        mesh=_mesh,
        interpret=interpret,
        compiler_params=pltpu.CompilerParams(needs_layout_passes=False),
        scratch_types=[
            pltpu.VMEM((N,), jnp.float32),
            pltpu.VMEM((N,), jnp.float32),
            pltpu.VMEM((PC,), jnp.int32),
            pltpu.VMEM((PC,), jnp.int32),
            pltpu.VMEM((PC,), jnp.float32),
        ],
    )


_deg_kernel = _mk_deg_kernel()
_agg_kernel = _mk_agg_kernel()
_pair_kernel = _mk_pair_kernel()


BLK = 1000
GRID = N // BLK


def _tc1_body(x_ref, w1_ref, b1_ref, dga_ref, dgb_ref, g1_ref, pre1_ref):
    h = jnp.dot(x_ref[...], w1_ref[...], preferred_element_type=jnp.float32)
    deg = dga_ref[...] + dgb_ref[...] + 1.0
    dinv = lax.rsqrt(deg)
    g1_ref[...] = h * dinv
    pre1_ref[...] = h * (dinv * dinv) + b1_ref[...]


def _tc2_body(a0_ref, a1_ref, dga_ref, dgb_ref, pre1_ref, w2_ref, b2_ref,
              g2_ref, pre2_ref):
    deg = dga_ref[...] + dgb_ref[...] + 1.0
    dinv = lax.rsqrt(deg)
    h1 = jnp.maximum((a0_ref[...] + a1_ref[...]) * dinv + pre1_ref[...], 0.0)
    h2 = jnp.dot(h1, w2_ref[...], preferred_element_type=jnp.float32)
    g2_ref[...] = h2 * dinv
    pre2_ref[...] = h2 * (dinv * dinv) + b2_ref[...]


def _tc3_body(a0_ref, a1_ref, dga_ref, dgb_ref, pre2_ref, wa_ref, wb_ref,
              blin_ref, s_ref, t_ref):
    deg = dga_ref[...] + dgb_ref[...] + 1.0
    dinv = lax.rsqrt(deg)
    h2 = (a0_ref[...] + a1_ref[...]) * dinv + pre2_ref[...]
    s_ref[...] = jnp.dot(h2, wa_ref[...],
                         preferred_element_type=jnp.float32) + blin_ref[...]
    t_ref[...] = jnp.dot(h2, wb_ref[...], preferred_element_type=jnp.float32)


def _row_spec(w):
    return pl.BlockSpec((BLK, w), lambda i: (i, 0))


def _full_spec(shape):
    return pl.BlockSpec(shape, lambda i: tuple(0 for _ in shape))


_tc1 = pl.pallas_call(
    _tc1_body,
    grid=(GRID,),
    in_specs=[_row_spec(H), _full_spec((H, H)), _full_spec((1, H)),
              _row_spec(1), _row_spec(1)],
    out_specs=[_row_spec(H), _row_spec(H)],
    out_shape=[jax.ShapeDtypeStruct((N, H), jnp.float32),
               jax.ShapeDtypeStruct((N, H), jnp.float32)],
)

_tc2 = pl.pallas_call(
    _tc2_body,
    grid=(GRID,),
    in_specs=[_row_spec(H), _row_spec(H), _row_spec(1), _row_spec(1),
              _row_spec(H), _full_spec((H, H)), _full_spec((1, H))],
    out_specs=[_row_spec(H), _row_spec(H)],
    out_shape=[jax.ShapeDtypeStruct((N, H), jnp.float32),
               jax.ShapeDtypeStruct((N, H), jnp.float32)],
)

_tc3 = pl.pallas_call(
    _tc3_body,
    grid=(GRID,),
    in_specs=[_row_spec(H), _row_spec(H), _row_spec(1), _row_spec(1),
              _row_spec(H), _full_spec((H, 1)), _full_spec((H, 1)),
              _full_spec((1, 1))],
    out_specs=[_row_spec(1), _row_spec(1)],
    out_shape=[jax.ShapeDtypeStruct((N, 1), jnp.float32),
               jax.ShapeDtypeStruct((N, 1), jnp.float32)],
)


def kernel(x, edge_index, edge_attr, pair_indices, W1, b1, W2, b2, Wlin, blin):
    del edge_attr
    src3 = edge_index[0].reshape(NW, NCHUNK, 1, CHUNK)
    dst3 = edge_index[1].reshape(NW, NCHUNK, 1, CHUNK)
    i0 = pair_indices[:, 0]
    i1 = pair_indices[:, 1]
    zeros_h = jnp.zeros((ZROW, H), jnp.float32)
    zeros_w = jnp.zeros((ZROW,), jnp.float32)
    ones_w = jnp.ones((CHUNK,), jnp.float32)

    degp = _deg_kernel(dst3, zeros_w, ones_w).reshape(NC, N)
    dga = degp[0, :, None]
    dgb = degp[1, :, None]

    g1, pre1 = _tc1(x, W1, b1.reshape(1, H), dga, dgb)
    agg1 = _agg_kernel(src3, dst3, g1, zeros_h)
    g2, pre2 = _tc2(agg1[0], agg1[1], dga, dgb, pre1, W2, b2.reshape(1, H))
    agg2 = _agg_kernel(src3, dst3, g2, zeros_h)
    s, t = _tc3(agg2[0], agg2[1], dga, dgb, pre2,
                Wlin[:H], Wlin[H:], blin.reshape(1, 1))
    return _pair_kernel(s.reshape(N), t.reshape(N), i0, i1)


# speedup vs baseline: 29.2637x; 1.0218x over previous
import functools

import jax
import jax.numpy as jnp
from jax import lax
from jax.experimental import pallas as pl
from jax.experimental.pallas import tpu as pltpu
from jax.experimental.pallas import tpu_sc as plsc

N = 10000
E = 320000
H = 128
P = 100000

NC = 2
NS = 16
NW = NC * NS
EPT = E // NW
CHUNK = 80
NCHUNK = EPT // CHUNK
ZROW = 640
LROW = N - (NS - 1) * ZROW
DEGW = 16
PC = 160
NPC = P // PC
PK = (NPC + NW - 1) // NW

_mesh = plsc.VectorSubcoreMesh(core_axis_name="c", subcore_axis_name="s")


def _zero_slice(zeros_hbm, acc_sh, s):
    @pl.when(s < NS - 1)
    def _():
        pltpu.sync_copy(zeros_hbm, acc_sh.at[pl.ds(s * ZROW, ZROW)])

    @pl.when(s == NS - 1)
    def _():
        pltpu.sync_copy(zeros_hbm.at[pl.ds(0, LROW)],
                        acc_sh.at[pl.ds((NS - 1) * ZROW, LROW)])


def _copy_out(acc_sh, out_hbm, c, s):
    @pl.when(s < NS - 1)
    def _():
        pltpu.sync_copy(acc_sh.at[pl.ds(s * ZROW, ZROW)],
                        out_hbm.at[c, pl.ds(s * ZROW, ZROW)])

    @pl.when(s == NS - 1)
    def _():
        pltpu.sync_copy(acc_sh.at[pl.ds((NS - 1) * ZROW, LROW)],
                        out_hbm.at[c, pl.ds((NS - 1) * ZROW, LROW)])


def _deg_body(dst_hbm, zeros_hbm, ones_hbm, out_hbm, idx_v, ones_v, stg_v,
              acc_sh):
    c = lax.axis_index("c")
    s = lax.axis_index("s")
    wid = c * NS + s
    pltpu.sync_copy(dst_hbm.at[wid], idx_v)
    pltpu.sync_copy(ones_hbm, ones_v)

    pltpu.sync_copy(zeros_hbm, stg_v)

    @pl.when(s < NS - 1)
    def _():
        pltpu.sync_copy(stg_v, acc_sh.at[pl.ds(s * ZROW, ZROW)])

    @pl.when(s == NS - 1)
    def _():
        pltpu.sync_copy(stg_v.at[pl.ds(0, LROW)],
                        acc_sh.at[pl.ds((NS - 1) * ZROW, LROW)])

    plsc.subcore_barrier()

    def body(ci, carry):
        pltpu.sync_copy(ones_v, acc_sh.at[idx_v.at[ci, 0]], add=True)
        return carry

    lax.fori_loop(0, NCHUNK, body, 0)
    plsc.subcore_barrier()

    @pl.when(s < NS - 1)
    def _():
        pltpu.sync_copy(acc_sh.at[pl.ds(s * ZROW, ZROW)], stg_v)
        pltpu.sync_copy(stg_v, out_hbm.at[pl.ds(c * N + s * ZROW, ZROW)])

    @pl.when(s == NS - 1)
    def _():
        pltpu.sync_copy(acc_sh.at[pl.ds((NS - 1) * ZROW, LROW)],
                        stg_v.at[pl.ds(0, LROW)])
        pltpu.sync_copy(stg_v.at[pl.ds(0, LROW)],
                        out_hbm.at[pl.ds(c * N + (NS - 1) * ZROW, LROW)])


NBUF = 4
NIDX = 2 * NBUF


def _agg_body(src_hbm, dst_hbm, g_hbm, zeros_hbm, out_hbm,
                sidx_v, didx_v, rows_v, acc_sh, semg, semis, semid):
    c = lax.axis_index("c")
    s = lax.axis_index("s")
    wid = c * NS + s
    _zero_slice(zeros_hbm, acc_sh, s)

    for j in range(NIDX):
        pltpu.async_copy(src_hbm.at[wid, j], sidx_v.at[j], semis.at[j])
    for b in range(NBUF):
        pltpu.async_copy(dst_hbm.at[wid, b], didx_v.at[b], semid.at[b])
        pltpu.make_async_copy(src_hbm.at[wid, 0], sidx_v.at[b],
                              semis.at[b]).wait()
        pltpu.async_copy(g_hbm.at[sidx_v.at[b, 0]], rows_v.at[b], semg.at[b])
    plsc.subcore_barrier()

    def body(ci, carry):
        b = lax.rem(ci, NBUF)
        pltpu.make_async_copy(dst_hbm.at[wid, 0], didx_v.at[b],
                              semid.at[b]).wait()
        pltpu.make_async_copy(g_hbm.at[sidx_v.at[0, 0]], rows_v.at[b],
                              semg.at[b]).wait()
        pltpu.sync_copy(rows_v.at[b], acc_sh.at[didx_v.at[b, 0]], add=True)
        nxt = ci + NBUF

        @pl.when(nxt < NCHUNK)
        def _():
            bn = lax.rem(nxt, NIDX)
            pltpu.make_async_copy(src_hbm.at[wid, 0], sidx_v.at[bn],
                                  semis.at[bn]).wait()
            pltpu.async_copy(g_hbm.at[sidx_v.at[bn, 0]], rows_v.at[b],
                             semg.at[b])
            pltpu.async_copy(dst_hbm.at[wid, nxt], didx_v.at[b], semid.at[b])

        nxt2 = ci + NIDX

        @pl.when(nxt2 < NCHUNK)
        def _():
            b2 = lax.rem(ci, NIDX)
            pltpu.async_copy(src_hbm.at[wid, nxt2], sidx_v.at[b2],
                             semis.at[b2])

        return carry

    lax.fori_loop(0, NCHUNK, body, 0)
    plsc.subcore_barrier()
    _copy_out(acc_sh, out_hbm, c, s)


def _pair_body(s_hbm, t_hbm, i0_hbm, i1_hbm, out_hbm,
                 stg_v, i0_v, i1_v, a_v, b_v, ob_v, s_sh, t_sh):
    c = lax.axis_index("c")
    s = lax.axis_index("s")
    wid = c * NS + s
    @pl.when(s < NS - 1)
    def _():
        pltpu.sync_copy(s_hbm.at[pl.ds(s * ZROW, ZROW)], stg_v)
        pltpu.sync_copy(stg_v, s_sh.at[pl.ds(s * ZROW, ZROW)])
        pltpu.sync_copy(t_hbm.at[pl.ds(s * ZROW, ZROW)], stg_v)
        pltpu.sync_copy(stg_v, t_sh.at[pl.ds(s * ZROW, ZROW)])

    @pl.when(s == NS - 1)
    def _():
        off = (NS - 1) * ZROW
        pltpu.sync_copy(s_hbm.at[pl.ds(off, LROW)], stg_v.at[pl.ds(0, LROW)])
        pltpu.sync_copy(stg_v.at[pl.ds(0, LROW)], s_sh.at[pl.ds(off, LROW)])
        pltpu.sync_copy(t_hbm.at[pl.ds(off, LROW)], stg_v.at[pl.ds(0, LROW)])
        pltpu.sync_copy(stg_v.at[pl.ds(0, LROW)], t_sh.at[pl.ds(off, LROW)])

    plsc.subcore_barrier()
    for k in range(PK):
        cid = wid + NW * k

        @pl.when(cid < NPC)
        def _():
            base = cid * PC
            pltpu.sync_copy(i0_hbm.at[pl.ds(base, PC)], i0_v)
            pltpu.sync_copy(i1_hbm.at[pl.ds(base, PC)], i1_v)
            pltpu.sync_copy(s_sh.at[i0_v], a_v)
            pltpu.sync_copy(t_sh.at[i1_v], b_v)

            def body(j, carry):
                z = a_v[pl.ds(j * 16, 16)] + b_v[pl.ds(j * 16, 16)]
                ob_v[pl.ds(j * 16, 16)] = 1.0 / (1.0 + jnp.exp(-z))
                return carry

            lax.fori_loop(0, PC // 16, body, 0)
            pltpu.sync_copy(ob_v, out_hbm.at[pl.ds(base, PC)])


def _mk_deg_kernel(interpret=False):
    return pl.kernel(
        _deg_body,
        out_type=jax.ShapeDtypeStruct((NC * N,), jnp.float32),
        mesh=_mesh,
        interpret=interpret,
        scratch_types=[
            pltpu.VMEM((NCHUNK, 1, CHUNK), jnp.int32),
            pltpu.VMEM((CHUNK,), jnp.float32),
            pltpu.VMEM((ZROW,), jnp.float32),
            pltpu.VMEM_SHARED((N,), jnp.float32),
        ],
    )


def _mk_agg_kernel(interpret=False):
    return pl.kernel(
        _agg_body,
        out_type=jax.ShapeDtypeStruct((NC, N, H), jnp.float32),
        mesh=_mesh,
        interpret=interpret,
        scratch_types=[
            pltpu.VMEM((NIDX, 1, CHUNK), jnp.int32),
            pltpu.VMEM((NBUF, 1, CHUNK), jnp.int32),
            pltpu.VMEM((NBUF, CHUNK, H), jnp.float32),
            pltpu.VMEM_SHARED((N, H), jnp.float32),
            pltpu.SemaphoreType.DMA((NBUF,)),
            pltpu.SemaphoreType.DMA((NIDX,)),
            pltpu.SemaphoreType.DMA((NBUF,)),
        ],
    )


def _mk_pair_kernel(interpret=False):
    return pl.kernel(
        _pair_body,
        out_type=jax.ShapeDtypeStruct((P,), jnp.float32),
        mesh=_mesh,
        interpret=interpret,
        scratch_types=[
            pltpu.VMEM((ZROW,), jnp.float32),
            pltpu.VMEM((PC,), jnp.int32),
            pltpu.VMEM((PC,), jnp.int32),
            pltpu.VMEM((PC,), jnp.float32),
            pltpu.VMEM((PC,), jnp.float32),
            pltpu.VMEM((PC,), jnp.float32),
            pltpu.VMEM_SHARED((N,), jnp.float32),
            pltpu.VMEM_SHARED((N,), jnp.float32),
        ],
    )


_deg_kernel = _mk_deg_kernel()
_agg_kernel = _mk_agg_kernel()
_pair_kernel = _mk_pair_kernel()


BLK = 1000
GRID = N // BLK


def _tc1_body(x_ref, w1_ref, b1_ref, dga_ref, dgb_ref, g1_ref, pre1_ref):
    h = jnp.dot(x_ref[...], w1_ref[...], preferred_element_type=jnp.float32)
    deg = dga_ref[...] + dgb_ref[...] + 1.0
    dinv = lax.rsqrt(deg)
    g1_ref[...] = h * dinv
    pre1_ref[...] = h * (dinv * dinv) + b1_ref[...]


def _tc2_body(a0_ref, a1_ref, dga_ref, dgb_ref, pre1_ref, w2_ref, b2_ref,
              g2_ref, pre2_ref):
    deg = dga_ref[...] + dgb_ref[...] + 1.0
    dinv = lax.rsqrt(deg)
    h1 = jnp.maximum((a0_ref[...] + a1_ref[...]) * dinv + pre1_ref[...], 0.0)
    h2 = jnp.dot(h1, w2_ref[...], preferred_element_type=jnp.float32)
    g2_ref[...] = h2 * dinv
    pre2_ref[...] = h2 * (dinv * dinv) + b2_ref[...]


def _tc3_body(a0_ref, a1_ref, dga_ref, dgb_ref, pre2_ref, wa_ref, wb_ref,
              blin_ref, s_ref, t_ref):
    deg = dga_ref[...] + dgb_ref[...] + 1.0
    dinv = lax.rsqrt(deg)
    h2 = (a0_ref[...] + a1_ref[...]) * dinv + pre2_ref[...]
    s_ref[...] = jnp.dot(h2, wa_ref[...],
                         preferred_element_type=jnp.float32) + blin_ref[...]
    t_ref[...] = jnp.dot(h2, wb_ref[...], preferred_element_type=jnp.float32)


def _row_spec(w):
    return pl.BlockSpec((BLK, w), lambda i: (i, 0))


def _full_spec(shape):
    return pl.BlockSpec(shape, lambda i: tuple(0 for _ in shape))


_tc1 = pl.pallas_call(
    _tc1_body,
    grid=(GRID,),
    in_specs=[_row_spec(H), _full_spec((H, H)), _full_spec((1, H)),
              _row_spec(1), _row_spec(1)],
    out_specs=[_row_spec(H), _row_spec(H)],
    out_shape=[jax.ShapeDtypeStruct((N, H), jnp.float32),
               jax.ShapeDtypeStruct((N, H), jnp.float32)],
)

_tc2 = pl.pallas_call(
    _tc2_body,
    grid=(GRID,),
    in_specs=[_row_spec(H), _row_spec(H), _row_spec(1), _row_spec(1),
              _row_spec(H), _full_spec((H, H)), _full_spec((1, H))],
    out_specs=[_row_spec(H), _row_spec(H)],
    out_shape=[jax.ShapeDtypeStruct((N, H), jnp.float32),
               jax.ShapeDtypeStruct((N, H), jnp.float32)],
)

_tc3 = pl.pallas_call(
    _tc3_body,
    grid=(GRID,),
    in_specs=[_row_spec(H), _row_spec(H), _row_spec(1), _row_spec(1),
              _row_spec(H), _full_spec((H, 1)), _full_spec((H, 1)),
              _full_spec((1, 1))],
    out_specs=[_row_spec(1), _row_spec(1)],
    out_shape=[jax.ShapeDtypeStruct((N, 1), jnp.float32),
               jax.ShapeDtypeStruct((N, 1), jnp.float32)],
)


def kernel(x, edge_index, edge_attr, pair_indices, W1, b1, W2, b2, Wlin, blin):
    del edge_attr
    src3 = edge_index[0].reshape(NW, NCHUNK, 1, CHUNK)
    dst3 = edge_index[1].reshape(NW, NCHUNK, 1, CHUNK)
    i0 = pair_indices[:, 0]
    i1 = pair_indices[:, 1]
    zeros_h = jnp.zeros((ZROW, H), jnp.float32)
    zeros_w = jnp.zeros((ZROW,), jnp.float32)
    ones_w = jnp.ones((CHUNK,), jnp.float32)

    degp = _deg_kernel(dst3, zeros_w, ones_w).reshape(NC, N)
    dga = degp[0, :, None]
    dgb = degp[1, :, None]

    g1, pre1 = _tc1(x, W1, b1.reshape(1, H), dga, dgb)
    agg1 = _agg_kernel(src3, dst3, g1, zeros_h)
    g2, pre2 = _tc2(agg1[0], agg1[1], dga, dgb, pre1, W2, b2.reshape(1, H))
    agg2 = _agg_kernel(src3, dst3, g2, zeros_h)
    s, t = _tc3(agg2[0], agg2[1], dga, dgb, pre2,
                Wlin[:H], Wlin[H:], blin.reshape(1, 1))
    return _pair_kernel(s.reshape(N), t.reshape(N), i0, i1)


# speedup vs baseline: 29.4736x; 1.0072x over previous
import functools

import jax
import jax.numpy as jnp
from jax import lax
from jax.experimental import pallas as pl
from jax.experimental.pallas import tpu as pltpu
from jax.experimental.pallas import tpu_sc as plsc

N = 10000
E = 320000
H = 128
P = 100000

NC = 2
NS = 16
NW = NC * NS
EPT = E // NW
CHUNK = 80
NCHUNK = EPT // CHUNK
ZROW = 640
LROW = N - (NS - 1) * ZROW
DEGW = 16
PC = 160
NPC = P // PC
PK = (NPC + NW - 1) // NW

_mesh = plsc.VectorSubcoreMesh(core_axis_name="c", subcore_axis_name="s")


def _zero_slice(zeros_hbm, acc_sh, s):
    @pl.when(s < NS - 1)
    def _():
        pltpu.sync_copy(zeros_hbm, acc_sh.at[pl.ds(s * ZROW, ZROW)])

    @pl.when(s == NS - 1)
    def _():
        pltpu.sync_copy(zeros_hbm.at[pl.ds(0, LROW)],
                        acc_sh.at[pl.ds((NS - 1) * ZROW, LROW)])


def _copy_out(acc_sh, out_hbm, c, s):
    @pl.when(s < NS - 1)
    def _():
        pltpu.sync_copy(acc_sh.at[pl.ds(s * ZROW, ZROW)],
                        out_hbm.at[c, pl.ds(s * ZROW, ZROW)])

    @pl.when(s == NS - 1)
    def _():
        pltpu.sync_copy(acc_sh.at[pl.ds((NS - 1) * ZROW, LROW)],
                        out_hbm.at[c, pl.ds((NS - 1) * ZROW, LROW)])


def _deg_body(ei_hbm, zeros_hbm, ones_hbm, out_hbm, idx_v, ones_v, stg_v,
              acc_sh):
    c = lax.axis_index("c")
    s = lax.axis_index("s")
    wid = c * NS + s
    pltpu.sync_copy(ei_hbm.at[1, wid], idx_v)
    pltpu.sync_copy(ones_hbm, ones_v)

    pltpu.sync_copy(zeros_hbm, stg_v)

    @pl.when(s < NS - 1)
    def _():
        pltpu.sync_copy(stg_v, acc_sh.at[pl.ds(s * ZROW, ZROW)])

    @pl.when(s == NS - 1)
    def _():
        pltpu.sync_copy(stg_v.at[pl.ds(0, LROW)],
                        acc_sh.at[pl.ds((NS - 1) * ZROW, LROW)])

    plsc.subcore_barrier()

    def body(ci, carry):
        pltpu.sync_copy(ones_v, acc_sh.at[idx_v.at[ci, 0]], add=True)
        return carry

    lax.fori_loop(0, NCHUNK, body, 0)
    plsc.subcore_barrier()

    @pl.when(s < NS - 1)
    def _():
        pltpu.sync_copy(acc_sh.at[pl.ds(s * ZROW, ZROW)], stg_v)
        pltpu.sync_copy(stg_v, out_hbm.at[pl.ds(c * N + s * ZROW, ZROW)])

    @pl.when(s == NS - 1)
    def _():
        pltpu.sync_copy(acc_sh.at[pl.ds((NS - 1) * ZROW, LROW)],
                        stg_v.at[pl.ds(0, LROW)])
        pltpu.sync_copy(stg_v.at[pl.ds(0, LROW)],
                        out_hbm.at[pl.ds(c * N + (NS - 1) * ZROW, LROW)])


NBUF = 4
NIDX = 2 * NBUF


def _agg_body(ei_hbm, g_hbm, zeros_hbm, out_hbm,
                sidx_v, didx_v, rows_v, acc_sh, semg, semis, semid):
    c = lax.axis_index("c")
    s = lax.axis_index("s")
    wid = c * NS + s
    _zero_slice(zeros_hbm, acc_sh, s)

    for j in range(NIDX):
        pltpu.async_copy(ei_hbm.at[0, wid, j], sidx_v.at[j], semis.at[j])
    for b in range(NBUF):
        pltpu.async_copy(ei_hbm.at[1, wid, b], didx_v.at[b], semid.at[b])
        pltpu.make_async_copy(ei_hbm.at[0, wid, 0], sidx_v.at[b],
                              semis.at[b]).wait()
        pltpu.async_copy(g_hbm.at[sidx_v.at[b, 0]], rows_v.at[b], semg.at[b])
    plsc.subcore_barrier()

    def body(ci, carry):
        b = lax.rem(ci, NBUF)
        pltpu.make_async_copy(ei_hbm.at[1, wid, 0], didx_v.at[b],
                              semid.at[b]).wait()
        pltpu.make_async_copy(g_hbm.at[sidx_v.at[0, 0]], rows_v.at[b],
                              semg.at[b]).wait()
        pltpu.sync_copy(rows_v.at[b], acc_sh.at[didx_v.at[b, 0]], add=True)
        nxt = ci + NBUF

        @pl.when(nxt < NCHUNK)
        def _():
            bn = lax.rem(nxt, NIDX)
            pltpu.make_async_copy(ei_hbm.at[0, wid, 0], sidx_v.at[bn],
                                  semis.at[bn]).wait()
            pltpu.async_copy(g_hbm.at[sidx_v.at[bn, 0]], rows_v.at[b],
                             semg.at[b])
            pltpu.async_copy(ei_hbm.at[1, wid, nxt], didx_v.at[b], semid.at[b])

        nxt2 = ci + NIDX

        @pl.when(nxt2 < NCHUNK)
        def _():
            b2 = lax.rem(ci, NIDX)
            pltpu.async_copy(ei_hbm.at[0, wid, nxt2], sidx_v.at[b2],
                             semis.at[b2])

        return carry

    lax.fori_loop(0, NCHUNK, body, 0)
    plsc.subcore_barrier()
    _copy_out(acc_sh, out_hbm, c, s)


def _pair_body(s_hbm, t_hbm, pf2_hbm, ev_hbm, od_hbm, out_hbm,
                 stg_v, f_v, ev_v, od_v, v_v, a_v, b_v, ob_v, st_sh, vs_sh):
    c = lax.axis_index("c")
    s = lax.axis_index("s")
    wid = c * NS + s
    @pl.when(s < NS - 1)
    def _():
        pltpu.sync_copy(s_hbm.at[pl.ds(s * ZROW, ZROW)], stg_v)
        pltpu.sync_copy(stg_v, st_sh.at[pl.ds(s * ZROW, ZROW)])
        pltpu.sync_copy(t_hbm.at[pl.ds(s * ZROW, ZROW)], stg_v)
        pltpu.sync_copy(stg_v, st_sh.at[pl.ds(N + s * ZROW, ZROW)])

    @pl.when(s == NS - 1)
    def _():
        off = (NS - 1) * ZROW
        pltpu.sync_copy(s_hbm.at[pl.ds(off, LROW)], stg_v.at[pl.ds(0, LROW)])
        pltpu.sync_copy(stg_v.at[pl.ds(0, LROW)], st_sh.at[pl.ds(off, LROW)])
        pltpu.sync_copy(t_hbm.at[pl.ds(off, LROW)], stg_v.at[pl.ds(0, LROW)])
        pltpu.sync_copy(stg_v.at[pl.ds(0, LROW)],
                        st_sh.at[pl.ds(N + off, LROW)])

    pltpu.sync_copy(ev_hbm.at[s], ev_v)
    pltpu.sync_copy(od_hbm.at[s], od_v)
    plsc.subcore_barrier()
    for k in range(PK):
        cid = wid + NW * k

        @pl.when(cid < NPC)
        def _():
            base = cid * PC
            pltpu.sync_copy(pf2_hbm.at[pl.ds(2 * base, 2 * PC)], f_v)
            pltpu.sync_copy(st_sh.at[f_v], v_v)
            pltpu.sync_copy(v_v, vs_sh.at[pl.ds(s * 2 * PC, 2 * PC)])
            pltpu.sync_copy(vs_sh.at[ev_v], a_v)
            pltpu.sync_copy(vs_sh.at[od_v], b_v)

            def body(j, carry):
                z = a_v[pl.ds(j * 16, 16)] + b_v[pl.ds(j * 16, 16)]
                ob_v[pl.ds(j * 16, 16)] = 1.0 / (1.0 + jnp.exp(-z))
                return carry

            lax.fori_loop(0, PC // 16, body, 0)
            pltpu.sync_copy(ob_v, out_hbm.at[pl.ds(base, PC)])


def _mk_deg_kernel(interpret=False):
    return pl.kernel(
        _deg_body,
        out_type=jax.ShapeDtypeStruct((NC * N,), jnp.float32),
        mesh=_mesh,
        interpret=interpret,
        scratch_types=[
            pltpu.VMEM((NCHUNK, 1, CHUNK), jnp.int32),
            pltpu.VMEM((CHUNK,), jnp.float32),
            pltpu.VMEM((ZROW,), jnp.float32),
            pltpu.VMEM_SHARED((N,), jnp.float32),
        ],
    )


def _mk_agg_kernel(interpret=False):
    return pl.kernel(
        _agg_body,
        out_type=jax.ShapeDtypeStruct((NC, N, H), jnp.float32),
        mesh=_mesh,
        interpret=interpret,
        scratch_types=[
            pltpu.VMEM((NIDX, 1, CHUNK), jnp.int32),
            pltpu.VMEM((NBUF, 1, CHUNK), jnp.int32),
            pltpu.VMEM((NBUF, CHUNK, H), jnp.float32),
            pltpu.VMEM_SHARED((N, H), jnp.float32),
            pltpu.SemaphoreType.DMA((NBUF,)),
            pltpu.SemaphoreType.DMA((NIDX,)),
            pltpu.SemaphoreType.DMA((NBUF,)),
        ],
    )


def _mk_pair_kernel(interpret=False):
    return pl.kernel(
        _pair_body,
        out_type=jax.ShapeDtypeStruct((P,), jnp.float32),
        mesh=_mesh,
        interpret=interpret,
        scratch_types=[
            pltpu.VMEM((ZROW,), jnp.float32),
            pltpu.VMEM((2 * PC,), jnp.int32),
            pltpu.VMEM((PC,), jnp.int32),
            pltpu.VMEM((PC,), jnp.int32),
            pltpu.VMEM((2 * PC,), jnp.float32),
            pltpu.VMEM((PC,), jnp.float32),
            pltpu.VMEM((PC,), jnp.float32),
            pltpu.VMEM((PC,), jnp.float32),
            pltpu.VMEM_SHARED((2 * N,), jnp.float32),
            pltpu.VMEM_SHARED((NS * 2 * PC,), jnp.float32),
        ],
    )


_deg_kernel = _mk_deg_kernel()
_agg_kernel = _mk_agg_kernel()
_pair_kernel = _mk_pair_kernel()


BLK = 1000
GRID = N // BLK


def _tc1_body(x_ref, w1_ref, b1_ref, dga_ref, dgb_ref, g1_ref, pre1_ref):
    h = jnp.dot(x_ref[...], w1_ref[...], preferred_element_type=jnp.float32)
    deg = dga_ref[...] + dgb_ref[...] + 1.0
    dinv = lax.rsqrt(deg)
    g1_ref[...] = h * dinv
    pre1_ref[...] = h * (dinv * dinv) + b1_ref[...]


def _tc2_body(a0_ref, a1_ref, dga_ref, dgb_ref, pre1_ref, w2_ref, b2_ref,
              g2_ref, pre2_ref):
    deg = dga_ref[...] + dgb_ref[...] + 1.0
    dinv = lax.rsqrt(deg)
    h1 = jnp.maximum((a0_ref[0] + a1_ref[0]) * dinv + pre1_ref[...], 0.0)
    h2 = jnp.dot(h1, w2_ref[...], preferred_element_type=jnp.float32)
    g2_ref[...] = h2 * dinv
    pre2_ref[...] = h2 * (dinv * dinv) + b2_ref[...]


def _tc3_body(a0_ref, a1_ref, dga_ref, dgb_ref, pre2_ref, wa_ref, wb_ref,
              blin_ref, s_ref, t_ref):
    deg = dga_ref[...] + dgb_ref[...] + 1.0
    dinv = lax.rsqrt(deg)
    h2 = (a0_ref[0] + a1_ref[0]) * dinv + pre2_ref[...]
    s_ref[...] = jnp.dot(h2, wa_ref[...],
                         preferred_element_type=jnp.float32) + blin_ref[...]
    t_ref[...] = jnp.dot(h2, wb_ref[...], preferred_element_type=jnp.float32)


def _row_spec(w):
    return pl.BlockSpec((BLK, w), lambda i: (i, 0))


def _full_spec(shape):
    return pl.BlockSpec(shape, lambda i: tuple(0 for _ in shape))


_tc1 = pl.pallas_call(
    _tc1_body,
    grid=(GRID,),
    in_specs=[_row_spec(H), _full_spec((H, H)), _full_spec((1, H)),
              _row_spec(1), _row_spec(1)],
    out_specs=[_row_spec(H), _row_spec(H)],
    out_shape=[jax.ShapeDtypeStruct((N, H), jnp.float32),
               jax.ShapeDtypeStruct((N, H), jnp.float32)],
)

_agg0_spec = pl.BlockSpec((1, BLK, H), lambda i: (0, i, 0))
_agg1_spec = pl.BlockSpec((1, BLK, H), lambda i: (1, i, 0))

_tc2 = pl.pallas_call(
    _tc2_body,
    grid=(GRID,),
    in_specs=[_agg0_spec, _agg1_spec, _row_spec(1), _row_spec(1),
              _row_spec(H), _full_spec((H, H)), _full_spec((1, H))],
    out_specs=[_row_spec(H), _row_spec(H)],
    out_shape=[jax.ShapeDtypeStruct((N, H), jnp.float32),
               jax.ShapeDtypeStruct((N, H), jnp.float32)],
)

_tc3 = pl.pallas_call(
    _tc3_body,
    grid=(GRID,),
    in_specs=[_agg0_spec, _agg1_spec, _row_spec(1), _row_spec(1),
              _row_spec(H), _full_spec((H, 1)), _full_spec((H, 1)),
              _full_spec((1, 1))],
    out_specs=[_row_spec(1), _row_spec(1)],
    out_shape=[jax.ShapeDtypeStruct((N, 1), jnp.float32),
               jax.ShapeDtypeStruct((N, 1), jnp.float32)],
)


def kernel(x, edge_index, edge_attr, pair_indices, W1, b1, W2, b2, Wlin, blin):
    del edge_attr
    ei = edge_index.reshape(2, NW, NCHUNK, 1, CHUNK)
    pf2 = pair_indices.reshape(2 * P) + jnp.tile(
        jnp.array([0, N], jnp.int32), P)
    ev = ((2 * PC * jnp.arange(NS, dtype=jnp.int32))[:, None]
          + 2 * jnp.arange(PC, dtype=jnp.int32)[None, :])
    od = ev + 1
    zeros_h = jnp.zeros((ZROW, H), jnp.float32)
    zeros_w = jnp.zeros((ZROW,), jnp.float32)
    ones_w = jnp.ones((CHUNK,), jnp.float32)

    degp = _deg_kernel(ei, zeros_w, ones_w).reshape(NC, N)
    dga = degp[0, :, None]
    dgb = degp[1, :, None]

    g1, pre1 = _tc1(x, W1, b1.reshape(1, H), dga, dgb)
    agg1 = _agg_kernel(ei, g1, zeros_h)
    g2, pre2 = _tc2(agg1, agg1, dga, dgb, pre1, W2, b2.reshape(1, H))
    agg2 = _agg_kernel(ei, g2, zeros_h)
    s, t = _tc3(agg2, agg2, dga, dgb, pre2,
                Wlin[:H], Wlin[H:], blin.reshape(1, 1))
    return _pair_kernel(s.reshape(N), t.reshape(N), pf2, ev, od)


# speedup vs baseline: 30.5585x; 1.0368x over previous
import functools

import jax
import jax.numpy as jnp
from jax import lax
from jax.experimental import pallas as pl
from jax.experimental.pallas import tpu as pltpu
from jax.experimental.pallas import tpu_sc as plsc

N = 10000
E = 320000
H = 128
P = 100000

NC = 2
NS = 16
NW = NC * NS
EPT = E // NW
CHUNK = 80
NCHUNK = EPT // CHUNK
ZROW = 640
LROW = N - (NS - 1) * ZROW
DEGW = 16
PC = 160
NPC = P // PC
PK = (NPC + NW - 1) // NW

_mesh = plsc.VectorSubcoreMesh(core_axis_name="c", subcore_axis_name="s")


def _zero_slice(zeros_hbm, acc_sh, s):
    @pl.when(s < NS - 1)
    def _():
        pltpu.sync_copy(zeros_hbm, acc_sh.at[pl.ds(s * ZROW, ZROW)])

    @pl.when(s == NS - 1)
    def _():
        pltpu.sync_copy(zeros_hbm.at[pl.ds(0, LROW)],
                        acc_sh.at[pl.ds((NS - 1) * ZROW, LROW)])


def _copy_out(acc_sh, out_hbm, c, s):
    @pl.when(s < NS - 1)
    def _():
        pltpu.sync_copy(acc_sh.at[pl.ds(s * ZROW, ZROW)],
                        out_hbm.at[c, pl.ds(s * ZROW, ZROW)])

    @pl.when(s == NS - 1)
    def _():
        pltpu.sync_copy(acc_sh.at[pl.ds((NS - 1) * ZROW, LROW)],
                        out_hbm.at[c, pl.ds((NS - 1) * ZROW, LROW)])


def _deg_body(ei_hbm, zeros_hbm, ones_hbm, out_hbm, idx_v, ones_v, stg_v,
              acc_sh):
    c = lax.axis_index("c")
    s = lax.axis_index("s")
    wid = c * NS + s
    pltpu.sync_copy(ei_hbm.at[1, wid], idx_v)
    pltpu.sync_copy(ones_hbm, ones_v)

    pltpu.sync_copy(zeros_hbm, stg_v)

    @pl.when(s < NS - 1)
    def _():
        pltpu.sync_copy(stg_v, acc_sh.at[pl.ds(s * ZROW, ZROW)])

    @pl.when(s == NS - 1)
    def _():
        pltpu.sync_copy(stg_v.at[pl.ds(0, LROW)],
                        acc_sh.at[pl.ds((NS - 1) * ZROW, LROW)])

    plsc.subcore_barrier()

    def body(ci, carry):
        pltpu.sync_copy(ones_v, acc_sh.at[idx_v.at[ci, 0]], add=True)
        return carry

    lax.fori_loop(0, NCHUNK, body, 0)
    plsc.subcore_barrier()

    @pl.when(s < NS - 1)
    def _():
        pltpu.sync_copy(acc_sh.at[pl.ds(s * ZROW, ZROW)], stg_v)
        pltpu.sync_copy(stg_v, out_hbm.at[pl.ds(c * N + s * ZROW, ZROW)])

    @pl.when(s == NS - 1)
    def _():
        pltpu.sync_copy(acc_sh.at[pl.ds((NS - 1) * ZROW, LROW)],
                        stg_v.at[pl.ds(0, LROW)])
        pltpu.sync_copy(stg_v.at[pl.ds(0, LROW)],
                        out_hbm.at[pl.ds(c * N + (NS - 1) * ZROW, LROW)])


NBUF = 4
NIDX = 2 * NBUF


def _agg_body(ei_hbm, g_hbm, zeros_hbm, out_hbm,
                sidx_v, didx_v, rows_v, acc_sh, semg, semis, semid):
    c = lax.axis_index("c")
    s = lax.axis_index("s")
    wid = c * NS + s
    _zero_slice(zeros_hbm, acc_sh, s)

    for j in range(NIDX):
        pltpu.async_copy(ei_hbm.at[0, wid, j], sidx_v.at[j], semis.at[j])
    for b in range(NBUF):
        pltpu.async_copy(ei_hbm.at[1, wid, b], didx_v.at[b], semid.at[b])
        pltpu.make_async_copy(ei_hbm.at[0, wid, 0], sidx_v.at[b],
                              semis.at[b]).wait()
        pltpu.async_copy(g_hbm.at[sidx_v.at[b, 0]], rows_v.at[b], semg.at[b])
    plsc.subcore_barrier()

    def body(ci, carry):
        b = lax.rem(ci, NBUF)
        pltpu.make_async_copy(ei_hbm.at[1, wid, 0], didx_v.at[b],
                              semid.at[b]).wait()
        pltpu.make_async_copy(g_hbm.at[sidx_v.at[0, 0]], rows_v.at[b],
                              semg.at[b]).wait()
        pltpu.sync_copy(rows_v.at[b], acc_sh.at[didx_v.at[b, 0]], add=True)
        nxt = ci + NBUF

        @pl.when(nxt < NCHUNK)
        def _():
            bn = lax.rem(nxt, NIDX)
            pltpu.make_async_copy(ei_hbm.at[0, wid, 0], sidx_v.at[bn],
                                  semis.at[bn]).wait()
            pltpu.async_copy(g_hbm.at[sidx_v.at[bn, 0]], rows_v.at[b],
                             semg.at[b])
            pltpu.async_copy(ei_hbm.at[1, wid, nxt], didx_v.at[b], semid.at[b])

        nxt2 = ci + NIDX

        @pl.when(nxt2 < NCHUNK)
        def _():
            b2 = lax.rem(ci, NIDX)
            pltpu.async_copy(ei_hbm.at[0, wid, nxt2], sidx_v.at[b2],
                             semis.at[b2])

        return carry

    lax.fori_loop(0, NCHUNK, body, 0)
    plsc.subcore_barrier()
    _copy_out(acc_sh, out_hbm, c, s)


def _pair_body(s_hbm, t_hbm, pf2_hbm, ev_hbm, od_hbm, out_hbm,
                 stg_v, f_v, ev_v, od_v, v_v, a_v, b_v, ob_v, st_sh, vs_sh):
    c = lax.axis_index("c")
    s = lax.axis_index("s")
    wid = c * NS + s
    @pl.when(s < NS - 1)
    def _():
        pltpu.sync_copy(s_hbm.at[pl.ds(s * ZROW, ZROW)], stg_v)
        pltpu.sync_copy(stg_v, st_sh.at[pl.ds(s * ZROW, ZROW)])
        pltpu.sync_copy(t_hbm.at[pl.ds(s * ZROW, ZROW)], stg_v)
        pltpu.sync_copy(stg_v, st_sh.at[pl.ds(N + s * ZROW, ZROW)])

    @pl.when(s == NS - 1)
    def _():
        off = (NS - 1) * ZROW
        pltpu.sync_copy(s_hbm.at[pl.ds(off, LROW)], stg_v.at[pl.ds(0, LROW)])
        pltpu.sync_copy(stg_v.at[pl.ds(0, LROW)], st_sh.at[pl.ds(off, LROW)])
        pltpu.sync_copy(t_hbm.at[pl.ds(off, LROW)], stg_v.at[pl.ds(0, LROW)])
        pltpu.sync_copy(stg_v.at[pl.ds(0, LROW)],
                        st_sh.at[pl.ds(N + off, LROW)])

    pltpu.sync_copy(ev_hbm.at[s], ev_v)
    pltpu.sync_copy(od_hbm.at[s], od_v)
    plsc.subcore_barrier()
    for k in range(PK):
        cid = wid + NW * k

        @pl.when(cid < NPC)
        def _():
            base = cid * PC
            pltpu.sync_copy(pf2_hbm.at[pl.ds(2 * base, 2 * PC)], f_v)
            pltpu.sync_copy(st_sh.at[f_v], v_v)
            pltpu.sync_copy(v_v, vs_sh.at[pl.ds(s * 2 * PC, 2 * PC)])
            pltpu.sync_copy(vs_sh.at[ev_v], a_v)
            pltpu.sync_copy(vs_sh.at[od_v], b_v)

            def body(j, carry):
                z = a_v[pl.ds(j * 16, 16)] + b_v[pl.ds(j * 16, 16)]
                ob_v[pl.ds(j * 16, 16)] = 1.0 / (1.0 + jnp.exp(-z))
                return carry

            lax.fori_loop(0, PC // 16, body, 0)
            pltpu.sync_copy(ob_v, out_hbm.at[pl.ds(base, PC)])


def _mk_deg_kernel(interpret=False):
    return pl.kernel(
        _deg_body,
        out_type=jax.ShapeDtypeStruct((NC * N,), jnp.float32),
        mesh=_mesh,
        interpret=interpret,
        scratch_types=[
            pltpu.VMEM((NCHUNK, 1, CHUNK), jnp.int32),
            pltpu.VMEM((CHUNK,), jnp.float32),
            pltpu.VMEM((ZROW,), jnp.float32),
            pltpu.VMEM_SHARED((N,), jnp.float32),
        ],
    )


def _mk_agg_kernel(interpret=False):
    return pl.kernel(
        _agg_body,
        out_type=jax.ShapeDtypeStruct((NC, N, H), jnp.float32),
        mesh=_mesh,
        interpret=interpret,
        scratch_types=[
            pltpu.VMEM((NIDX, 1, CHUNK), jnp.int32),
            pltpu.VMEM((NBUF, 1, CHUNK), jnp.int32),
            pltpu.VMEM((NBUF, CHUNK, H), jnp.float32),
            pltpu.VMEM_SHARED((N, H), jnp.float32),
            pltpu.SemaphoreType.DMA((NBUF,)),
            pltpu.SemaphoreType.DMA((NIDX,)),
            pltpu.SemaphoreType.DMA((NBUF,)),
        ],
    )


def _mk_pair_kernel(interpret=False):
    return pl.kernel(
        _pair_body,
        out_type=jax.ShapeDtypeStruct((P,), jnp.float32),
        mesh=_mesh,
        interpret=interpret,
        scratch_types=[
            pltpu.VMEM((ZROW,), jnp.float32),
            pltpu.VMEM((2 * PC,), jnp.int32),
            pltpu.VMEM((PC,), jnp.int32),
            pltpu.VMEM((PC,), jnp.int32),
            pltpu.VMEM((2 * PC,), jnp.float32),
            pltpu.VMEM((PC,), jnp.float32),
            pltpu.VMEM((PC,), jnp.float32),
            pltpu.VMEM((PC,), jnp.float32),
            pltpu.VMEM_SHARED((2 * N,), jnp.float32),
            pltpu.VMEM_SHARED((NS * 2 * PC,), jnp.float32),
        ],
    )


_deg_kernel = _mk_deg_kernel()
_agg_kernel = _mk_agg_kernel()
_pair_kernel = _mk_pair_kernel()


BLK = 1000
GRID = N // BLK


def _tc1_body(x_ref, w1_ref, b1_ref, dga_ref, dgb_ref, g1_ref, pre1_ref):
    h = jnp.dot(x_ref[...], w1_ref[...], preferred_element_type=jnp.float32)
    deg = dga_ref[...] + dgb_ref[...] + 1.0
    dinv = lax.rsqrt(deg)
    g1_ref[...] = h * dinv
    pre1_ref[...] = h * (dinv * dinv) + b1_ref[...]


def _tc2_body(a0_ref, a1_ref, dga_ref, dgb_ref, pre1_ref, w2_ref, b2_ref,
              g2_ref, pre2_ref):
    deg = dga_ref[...] + dgb_ref[...] + 1.0
    dinv = lax.rsqrt(deg)
    h1 = jnp.maximum((a0_ref[0] + a1_ref[0]) * dinv + pre1_ref[...], 0.0)
    h2 = jnp.dot(h1, w2_ref[...], preferred_element_type=jnp.float32)
    g2_ref[...] = h2 * dinv
    pre2_ref[...] = h2 * (dinv * dinv) + b2_ref[...]


def _tc3_body(agg_ref, dga_ref, dgb_ref, pre2_ref, wa_ref, wb_ref,
              blin_ref, s_ref, t_ref):
    deg = dga_ref[...] + dgb_ref[...] + 1.0
    dinv = lax.rsqrt(deg)
    h2 = (agg_ref[0] + agg_ref[1]) * dinv + pre2_ref[...]
    dn = (((0,), (1,)), ((), ()))
    s_ref[...] = lax.dot_general(
        wa_ref[...], h2, dn,
        preferred_element_type=jnp.float32) + blin_ref[...]
    t_ref[...] = lax.dot_general(
        wb_ref[...], h2, dn, preferred_element_type=jnp.float32)


def _row_spec(w):
    return pl.BlockSpec((BLK, w), lambda i: (i, 0))


def _full_spec(shape):
    return pl.BlockSpec(shape, lambda i: tuple(0 for _ in shape))


_tc1 = pl.pallas_call(
    _tc1_body,
    grid=(GRID,),
    in_specs=[_row_spec(H), _full_spec((H, H)), _full_spec((1, H)),
              _row_spec(1), _row_spec(1)],
    out_specs=[_row_spec(H), _row_spec(H)],
    out_shape=[jax.ShapeDtypeStruct((N, H), jnp.float32),
               jax.ShapeDtypeStruct((N, H), jnp.float32)],
)

_agg0_spec = pl.BlockSpec((1, BLK, H), lambda i: (0, i, 0))
_agg1_spec = pl.BlockSpec((1, BLK, H), lambda i: (1, i, 0))

_tc2 = pl.pallas_call(
    _tc2_body,
    grid=(GRID,),
    in_specs=[_agg0_spec, _agg1_spec, _row_spec(1), _row_spec(1),
              _row_spec(H), _full_spec((H, H)), _full_spec((1, H))],
    out_specs=[_row_spec(H), _row_spec(H)],
    out_shape=[jax.ShapeDtypeStruct((N, H), jnp.float32),
               jax.ShapeDtypeStruct((N, H), jnp.float32)],
)

_tc3 = pl.pallas_call(
    _tc3_body,
    out_shape=[jax.ShapeDtypeStruct((1, N), jnp.float32),
               jax.ShapeDtypeStruct((1, N), jnp.float32)],
)


def kernel(x, edge_index, edge_attr, pair_indices, W1, b1, W2, b2, Wlin, blin):
    del edge_attr
    ei = edge_index.reshape(2, NW, NCHUNK, 1, CHUNK)
    pf2 = pair_indices.reshape(2 * P) + jnp.tile(
        jnp.array([0, N], jnp.int32), P)
    ev = ((2 * PC * jnp.arange(NS, dtype=jnp.int32))[:, None]
          + 2 * jnp.arange(PC, dtype=jnp.int32)[None, :])
    od = ev + 1
    zeros_h = jnp.zeros((ZROW, H), jnp.float32)
    zeros_w = jnp.zeros((ZROW,), jnp.float32)
    ones_w = jnp.ones((CHUNK,), jnp.float32)

    degp = _deg_kernel(ei, zeros_w, ones_w).reshape(NC, N)
    dga = degp[0, :, None]
    dgb = degp[1, :, None]

    g1, pre1 = _tc1(x, W1, b1.reshape(1, H), dga, dgb)
    agg1 = _agg_kernel(ei, g1, zeros_h)
    g2, pre2 = _tc2(agg1, agg1, dga, dgb, pre1, W2, b2.reshape(1, H))
    agg2 = _agg_kernel(ei, g2, zeros_h)
    s, t = _tc3(agg2, dga, dgb, pre2,
                Wlin[:H], Wlin[H:], blin.reshape(1, 1))
    return _pair_kernel(s.reshape(N), t.reshape(N), pf2, ev, od)
